# Initial kernel scaffold; baseline (speedup 1.0000x reference)
import jax
import jax.numpy as jnp
from jax.experimental import pallas as pl


def kernel(x, tables):
    raise NotImplementedError("write your pallas kernel here")


# speedup vs baseline: 1.1912x; 1.1912x over previous
import functools

import jax
import jax.numpy as jnp
from jax import lax
from jax.experimental import pallas as pl
from jax.experimental.pallas import tpu as pltpu
from jax.experimental.pallas import tpu_sc as plsc

NUM_FIELDS = 26
VOCAB = 100000
EMB = 32
BATCH = 16384

NC = 2
NS = 16
NW = NC * NS
ROWS_PER_W = BATCH // NW
CHUNK = 64
N_CHUNKS = ROWS_PER_W // CHUNK
LANES = 16


def _make_kernel():
    mesh = plsc.VectorSubcoreMesh(core_axis_name="c", subcore_axis_name="s")

    @functools.partial(
        pl.kernel,
        out_type=jax.ShapeDtypeStruct((BATCH, EMB), jnp.float32),
        mesh=mesh,
        compiler_params=pltpu.CompilerParams(use_tc_tiling_on_sc=False),
        scratch_types=[
            pltpu.VMEM((NUM_FIELDS, ROWS_PER_W), jnp.int32),
            pltpu.VMEM((NUM_FIELDS, CHUNK, EMB), jnp.float32),
            pltpu.VMEM((CHUNK, EMB), jnp.float32),
            pltpu.SemaphoreType.DMA,
        ],
    )
    def k(ftab_hbm, xw_hbm, out_hbm, idx_v, buf, acc, sem):
        wid = lax.axis_index("s") * NC + lax.axis_index("c")
        wbase = wid * ROWS_PER_W

        pltpu.sync_copy(xw_hbm.at[wid], idx_v)

        @pl.loop(0, ROWS_PER_W // LANES)
        def _offsets(c):
            sl = pl.ds(c * LANES, LANES)
            for f in range(1, NUM_FIELDS):
                plsc.addupdate(
                    idx_v.at[f, sl],
                    jnp.full((LANES,), f * VOCAB, dtype=jnp.int32),
                )

        @pl.loop(0, N_CHUNKS)
        def _chunk(g):
            base = pl.multiple_of(g * CHUNK, CHUNK)
            copies = []
            for f in range(NUM_FIELDS):
                copies.append(
                    pltpu.async_copy(
                        ftab_hbm.at[idx_v.at[f, pl.ds(base, CHUNK)]],
                        buf.at[f],
                        sem,
                    )
                )
            for c in copies:
                c.wait()

            @pl.loop(0, CHUNK)
            def _row(r):
                for half in range(EMB // LANES):
                    sl = pl.ds(half * LANES, LANES)
                    t = buf[0, r, sl]
                    for f in range(1, NUM_FIELDS):
                        t = t + buf[f, r, sl]
                    acc[r, sl] = t

            pltpu.sync_copy(
                acc, out_hbm.at[pl.ds(pl.multiple_of(wbase + base, CHUNK), CHUNK)]
            )

    return k


_sc_kernel = _make_kernel()


@jax.jit
def kernel(x, tables):
    ftab = tables.reshape(NUM_FIELDS * VOCAB, EMB)
    xw = (
        x.astype(jnp.int32)
        .reshape(NW, ROWS_PER_W, NUM_FIELDS)
        .transpose(0, 2, 1)
    )
    return _sc_kernel(ftab, xw)
